# baseline (speedup 1.0000x reference)
import functools

import jax
import jax.numpy as jnp
from jax import lax
from jax.experimental import pallas as pl
from jax.experimental.pallas import tpu as pltpu
from jax.experimental.pallas import tpu_sc as plsc

N_V = 10000
N_E = 10000
N_PAIRS = 320000
D = 128
FH = 64
NC = 2
NS = 16
CHUNK = 80
NCHUNK = N_PAIRS // NS // CHUNK
N_SEG = 10240
ROWS_PER_TILE = N_SEG // NS
NROW = 3
NIDX = 6
STEADY0 = 6
TAIL = 4
STEADYN = (NCHUNK - STEADY0 - TAIL) // NIDX
EROWS = N_E // NS // 5


def _sc_body(table, gidx1, gidx2, sidx1, sidx2, zf, zd, out,
             acc1, dacc, acc2, idx_g, idx_s, rows, ones, cbuf, dbuf, cobuf,
             *sems):
    sem_i = sems[:NIDX]
    sem_g = sems[NIDX:NIDX + NROW]
    sem_s = sems[NIDX + NROW:]
    c = lax.axis_index("c")
    s = lax.axis_index("s")
    tile_rows = pl.ds(s * ROWS_PER_TILE, ROWS_PER_TILE)

    pltpu.sync_copy(zf, acc1.at[tile_rows])
    pltpu.sync_copy(zd, dacc.at[tile_rows])
    pltpu.sync_copy(zf, acc2.at[tile_rows])

    def ones_body(r, _):
        ones[r, pl.ds(0, 16)] = jnp.ones((16,), jnp.float32)
        return 0
    lax.fori_loop(0, CHUNK, ones_body, 0)
    plsc.subcore_barrier()

    def stream_phase(src, gidx_row, sidx_row, accf):
        def start_idx(t, bi):
            pltpu.async_copy(gidx_row(t), idx_g.at[bi], sem_i[bi])
            pltpu.async_copy(sidx_row(t), idx_s.at[bi], sem_i[bi])

        def wait_idx(bi):
            pltpu.make_async_copy(gidx_row(0), idx_g.at[bi], sem_i[bi]).wait()
            pltpu.make_async_copy(sidx_row(0), idx_s.at[bi], sem_i[bi]).wait()

        def start_gather(bi, br):
            pltpu.async_copy(src.at[idx_g.at[bi]], rows.at[br], sem_g[br])

        def wait_gather(bi, br):
            pltpu.make_async_copy(src.at[idx_g.at[bi]], rows.at[br],
                                  sem_g[br]).wait()

        def start_scatter(bi, br):
            pltpu.async_copy(rows.at[br], accf.at[idx_s.at[bi]], sem_s[br],
                             add=True)
            pltpu.async_copy(ones, dacc.at[idx_s.at[bi]], sem_s[br], add=True)

        def wait_scatter(bi, br):
            pltpu.make_async_copy(rows.at[br], accf.at[idx_s.at[bi]],
                                  sem_s[br]).wait()
            pltpu.make_async_copy(ones, dacc.at[idx_s.at[bi]],
                                  sem_s[br]).wait()

        def step(t, bi, br, wait_s=True, retire=True, prefetch=True):
            wait_idx(bi)
            if wait_s:
                wait_scatter(bi, br)
            start_gather(bi, br)
            if prefetch:
                start_idx(t + 1, (bi + 1) % NIDX)
            if retire:
                obi = (bi - 1) % NIDX
                obr = (br - 1) % NROW
                wait_gather(obi, obr)
                start_scatter(obi, obr)

        start_idx(0, 0)
        for t in range(STEADY0):
            step(t, t % NIDX, t % NROW, wait_s=(t >= NROW), retire=(t >= 1))

        def group_body(g, _):
            for j in range(NIDX):
                t = STEADY0 + g * NIDX + j
                step(t, (STEADY0 + j) % NIDX, (STEADY0 + j) % NROW)
            return 0
        lax.fori_loop(0, STEADYN, group_body, 0)

        for t in range(NCHUNK - TAIL, NCHUNK):
            step(t, t % NIDX, t % NROW, prefetch=(t < NCHUNK - 1))

        tl = NCHUNK - 1
        wait_gather(tl % NIDX, tl % NROW)
        start_scatter(tl % NIDX, tl % NROW)
        for b in range(NROW):
            wait_scatter(0, b)

    def epilogue(accf, final):
        base = s * (5 * EROWS)

        def blk_body(i, _):
            r0 = base + i * EROWS
            pltpu.sync_copy(accf.at[pl.ds(r0, EROWS)], cbuf)
            pltpu.sync_copy(dacc.at[pl.ds(r0, EROWS)], dbuf)

            def row_body(r5, _):
                for u in range(5):
                    r = r5 * 5 + u
                    deg = jnp.maximum(dbuf[r, pl.ds(0, 16)], 1.0)
                    inv = 1.0 / deg
                    for j in range(FH // 16):
                        v = cbuf[r, pl.ds(16 * j, 16)] * inv
                        if final:
                            v = jnp.maximum(v, 0.0)
                        cobuf[r, pl.ds(16 * j, 16)] = v
                return 0
            lax.fori_loop(0, EROWS // 5, row_body, 0)

            if final:
                pltpu.sync_copy(cobuf,
                                out.at[pl.ds(r0, EROWS), pl.ds(c * FH, FH)])
            else:
                pltpu.sync_copy(cobuf, accf.at[pl.ds(r0, EROWS)])
            return 0
        lax.fori_loop(0, 5, blk_body, 0)

    stream_phase(table,
                 lambda t: gidx1.at[c, s, t],
                 lambda t: sidx1.at[s, t],
                 acc1)
    plsc.subcore_barrier()
    epilogue(acc1, final=False)
    pltpu.sync_copy(zd, dacc.at[tile_rows])
    plsc.subcore_barrier()

    stream_phase(acc1,
                 lambda t: gidx2.at[s, t],
                 lambda t: sidx2.at[s, t],
                 acc2)
    plsc.subcore_barrier()
    epilogue(acc2, final=True)


@functools.lru_cache(maxsize=1)
def _make_sc_kernel():
    mesh = plsc.VectorSubcoreMesh(core_axis_name="c", subcore_axis_name="s",
                                  num_cores=NC, num_subcores=NS)
    return pl.kernel(
        _sc_body,
        out_type=jax.ShapeDtypeStruct((N_V, D), jnp.float32),
        mesh=mesh,
        scratch_types=[
            pltpu.VMEM_SHARED((N_SEG, FH), jnp.float32),
            pltpu.VMEM_SHARED((N_SEG, 16), jnp.float32),
            pltpu.VMEM_SHARED((N_SEG, FH), jnp.float32),
            pltpu.VMEM((NIDX, CHUNK), jnp.int32),
            pltpu.VMEM((NIDX, CHUNK), jnp.int32),
            pltpu.VMEM((NROW, CHUNK, FH), jnp.float32),
            pltpu.VMEM((CHUNK, 16), jnp.float32),
            pltpu.VMEM((EROWS, FH), jnp.float32),
            pltpu.VMEM((EROWS, 16), jnp.float32),
            pltpu.VMEM((EROWS, FH), jnp.float32),
            *([pltpu.SemaphoreType.DMA] * (NIDX + 2 * NROW)),
        ],
        compiler_params=pltpu.CompilerParams(use_tc_tiling_on_sc=False),
    )


_BR = 1000


def _matmul_body(x_ref, w_ref, b_ref, o_ref):
    x = x_ref[...]
    w = w_ref[...]
    y = lax.dot_general(x, w, (((1,), (1,)), ((), ())),
                        preferred_element_type=jnp.float32)
    y = y + b_ref[...]
    o_ref[...] = jnp.stack([y[:, :FH], y[:, FH:]], axis=0)


def kernel(X, pair_v, pair_e, W, b, group_weight):
    del group_weight

    xt2 = pl.pallas_call(
        _matmul_body,
        grid=(N_V // _BR,),
        in_specs=[
            pl.BlockSpec((_BR, D), lambda i: (i, 0)),
            pl.BlockSpec((D, D), lambda i: (0, 0)),
            pl.BlockSpec((1, D), lambda i: (0, 0)),
        ],
        out_specs=pl.BlockSpec((NC, _BR, FH), lambda i: (0, i, 0)),
        out_shape=jax.ShapeDtypeStruct((NC, N_V, FH), jnp.float32),
    )(X, W, b.reshape(1, D))

    pvg = jnp.stack([pair_v, pair_v + N_V]).reshape(NC, NS, NCHUNK, CHUNK)
    peg = pair_e.reshape(NS, NCHUNK, CHUNK)
    pvs = pair_v.reshape(NS, NCHUNK, CHUNK)
    zf = jnp.zeros((ROWS_PER_TILE, FH), jnp.float32)
    zd = jnp.zeros((ROWS_PER_TILE, 16), jnp.float32)

    return _make_sc_kernel()(xt2.reshape(NC * N_V, FH), pvg, peg, peg, pvs,
                             zf, zd)


# speedup vs baseline: 1.0925x; 1.0925x over previous
import functools

import jax
import jax.numpy as jnp
from jax import lax
from jax.experimental import pallas as pl
from jax.experimental.pallas import tpu as pltpu
from jax.experimental.pallas import tpu_sc as plsc

N_V = 10000
N_E = 10000
N_PAIRS = 320000
D = 128
DA = 144
NC = 2
NS = 16
NW = NC * NS
PAIRS_PER_W = N_PAIRS // NW
CHUNK = 80
NCHUNK = PAIRS_PER_W // CHUNK
N_SEG = 10240
ROWS_PER_TILE = N_SEG // NS
NROW = 3
NIDX = 6
STEADY0 = 6
TAIL = 5
STEADYN = (NCHUNK - STEADY0 - TAIL) // NIDX


@functools.lru_cache(maxsize=1)
def _make_sc_segment_pass():
    mesh = plsc.VectorSubcoreMesh(core_axis_name="c", subcore_axis_name="s",
                                  num_cores=NC, num_subcores=NS)
    return pl.kernel(
        _sc_segment_body,
        out_type=jax.ShapeDtypeStruct((NC, N_SEG, DA), jnp.float32),
        mesh=mesh,
        scratch_types=[
            pltpu.VMEM_SHARED((N_SEG, DA), jnp.float32),
            pltpu.VMEM((NIDX, CHUNK), jnp.int32),
            pltpu.VMEM((NIDX, CHUNK), jnp.int32),
            pltpu.VMEM((NROW, CHUNK, DA), jnp.float32),
            *([pltpu.SemaphoreType.DMA] * (NIDX + 2 * NROW)),
        ],
        compiler_params=pltpu.CompilerParams(use_tc_tiling_on_sc=False),
    )


def _sc_segment_body(table, gidx, sidx, zeros, out, acc, idx_g, idx_s, rows, *sems):
    sem_i = sems[:NIDX]
    sem_g = sems[NIDX:NIDX + NROW]
    sem_s = sems[NIDX + NROW:]
    c = lax.axis_index("c")
    s = lax.axis_index("s")
    wid = c * NS + s

    pltpu.sync_copy(zeros, acc.at[pl.ds(s * ROWS_PER_TILE, ROWS_PER_TILE)])
    plsc.subcore_barrier()

    def start_idx(t, bi):
        pltpu.async_copy(gidx.at[wid, t], idx_g.at[bi], sem_i[bi])
        pltpu.async_copy(sidx.at[wid, t], idx_s.at[bi], sem_i[bi])

    def wait_idx(bi):
        pltpu.make_async_copy(gidx.at[0, 0], idx_g.at[bi], sem_i[bi]).wait()
        pltpu.make_async_copy(sidx.at[0, 0], idx_s.at[bi], sem_i[bi]).wait()

    def start_gather(bi, br):
        pltpu.async_copy(table.at[idx_g.at[bi]], rows.at[br], sem_g[br])

    def wait_gather(bi, br):
        pltpu.make_async_copy(table.at[idx_g.at[bi]], rows.at[br], sem_g[br]).wait()

    def start_scatter(bi, br):
        pltpu.async_copy(rows.at[br], acc.at[idx_s.at[bi]], sem_s[br], add=True)

    def wait_scatter(bi, br):
        pltpu.make_async_copy(rows.at[br], acc.at[idx_s.at[bi]], sem_s[br]).wait()

    def step(t, bi, br, wait_s=True, retire=True, prefetch=True):
        wait_idx(bi)
        if wait_s:
            wait_scatter(bi, br)
        start_gather(bi, br)
        if prefetch:
            start_idx(t + 1, (bi + 1) % NIDX)
        if retire:
            obi = (bi - 1) % NIDX
            obr = (br - 1) % NROW
            wait_gather(obi, obr)
            start_scatter(obi, obr)

    start_idx(0, 0)
    for t in range(STEADY0):
        step(t, t % NIDX, t % NROW, wait_s=(t >= NROW), retire=(t >= 1))

    def group_body(g, _):
        for j in range(NIDX):
            t = STEADY0 + g * NIDX + j
            step(t, (STEADY0 + j) % NIDX, (STEADY0 + j) % NROW)
        return 0
    lax.fori_loop(0, STEADYN, group_body, 0)

    for t in range(NCHUNK - TAIL, NCHUNK):
        step(t, t % NIDX, t % NROW, prefetch=(t < NCHUNK - 1))

    tl = NCHUNK - 1
    wait_gather(tl % NIDX, tl % NROW)
    start_scatter(tl % NIDX, tl % NROW)
    for b in range(NROW):
        wait_scatter(0, b)

    plsc.subcore_barrier()
    pltpu.sync_copy(acc.at[pl.ds(s * ROWS_PER_TILE, ROWS_PER_TILE)],
                    out.at[c, pl.ds(s * ROWS_PER_TILE, ROWS_PER_TILE)])


_BR = 1000


def _matmul_body(x_ref, w_ref, b_ref, o_ref):
    x = x_ref[...]
    w = w_ref[...]
    y = lax.dot_general(x, w, (((1,), (1,)), ((), ())),
                        preferred_element_type=jnp.float32)
    y = y + b_ref[...]
    col = lax.broadcasted_iota(jnp.int32, (_BR, DA), 1)
    o_ref[...] = jnp.where(col < D, jnp.pad(y, ((0, 0), (0, DA - D))),
                           jnp.where(col == D, 1.0, 0.0))


def _combine_body(p_ref, o_ref):
    p = p_ref[...]
    ssum = p[0] + p[1]
    deg = jnp.maximum(ssum[:, D:D + 1], 1.0)
    col = lax.broadcasted_iota(jnp.int32, (_BR, DA), 1)
    o_ref[...] = jnp.where(col < D, ssum / deg,
                           jnp.where(col == D, 1.0, 0.0))


def _finalize_body(p_ref, o_ref):
    p = p_ref[...]
    ssum = p[0] + p[1]
    deg = jnp.maximum(ssum[:, D:D + 1], 1.0)
    o_ref[...] = jnp.maximum(ssum[:, :D] / deg, 0.0)


def kernel(X, pair_v, pair_e, W, b, group_weight):
    del group_weight

    aug = pl.pallas_call(
        _matmul_body,
        grid=(N_V // _BR,),
        in_specs=[
            pl.BlockSpec((_BR, D), lambda i: (i, 0)),
            pl.BlockSpec((D, D), lambda i: (0, 0)),
            pl.BlockSpec((1, D), lambda i: (0, 0)),
        ],
        out_specs=pl.BlockSpec((_BR, DA), lambda i: (i, 0)),
        out_shape=jax.ShapeDtypeStruct((N_V, DA), jnp.float32),
    )(X, W, b.reshape(1, D))

    pv3 = pair_v.reshape(NW, NCHUNK, CHUNK)
    pe3 = pair_e.reshape(NW, NCHUNK, CHUNK)
    zeros = jnp.zeros((ROWS_PER_TILE, DA), jnp.float32)

    sc_pass = _make_sc_segment_pass()
    partial_e = sc_pass(aug, pv3, pe3, zeros)

    e_feat_aug = pl.pallas_call(
        _combine_body,
        grid=(N_E // _BR,),
        in_specs=[pl.BlockSpec((NC, _BR, DA), lambda i: (0, i, 0))],
        out_specs=pl.BlockSpec((_BR, DA), lambda i: (i, 0)),
        out_shape=jax.ShapeDtypeStruct((N_E, DA), jnp.float32),
    )(partial_e)

    partial_v = sc_pass(e_feat_aug, pe3, pv3, zeros)

    out = pl.pallas_call(
        _finalize_body,
        grid=(N_V // _BR,),
        in_specs=[pl.BlockSpec((NC, _BR, DA), lambda i: (0, i, 0))],
        out_specs=pl.BlockSpec((_BR, D), lambda i: (i, 0)),
        out_shape=jax.ShapeDtypeStruct((N_V, D), jnp.float32),
    )(partial_v)

    return out
